# Initial kernel scaffold; baseline (speedup 1.0000x reference)
import jax
import jax.numpy as jnp
from jax.experimental import pallas as pl


def kernel(x):
    raise NotImplementedError("write your pallas kernel here")


# speedup vs baseline: 43.8185x; 43.8185x over previous
import jax
import jax.numpy as jnp
from jax.experimental import pallas as pl
from jax.experimental.pallas import tpu as pltpu

KEEP = 0.1


def _tc_body(k, x_ref, o_ref):
    x = x_ref[...]
    u = jax.lax.bitcast_convert_type(x, jnp.int32) & jnp.int32(0x7FFFFFFF)
    B = x.shape[0]

    def step(_, lohi):
        lo, hi = lohi
        mid = lo + ((hi - lo + jnp.int32(1)) >> 1)
        cnt = jnp.sum((u >= mid).astype(jnp.int32), axis=1, keepdims=True)
        ge = cnt >= k
        return jnp.where(ge, mid, lo), jnp.where(ge, hi, mid - 1)

    lo0 = jnp.zeros((B, 1), jnp.int32)
    hi0 = jnp.full((B, 1), 0x7F800000, jnp.int32)
    lo, _ = jax.lax.fori_loop(0, 31, step, (lo0, hi0))
    o_ref[...] = jnp.where(u >= lo, x, jnp.float32(0.0))


def kernel(x):
    B, N = x.shape
    k = max(1, int(N * KEEP))
    import functools
    return pl.pallas_call(
        functools.partial(_tc_body, k),
        out_shape=jax.ShapeDtypeStruct((B, N), x.dtype),
        in_specs=[pl.BlockSpec(memory_space=pltpu.VMEM)],
        out_specs=pl.BlockSpec(memory_space=pltpu.VMEM),
    )(x)
